# baseline (speedup 1.0000x reference)
import functools

import jax
import jax.numpy as jnp
from jax import lax
from jax.experimental import pallas as pl
from jax.experimental.pallas import tpu as pltpu
from jax.experimental.pallas import tpu_sc as plsc

NC = 2
NS = 16
NW = NC * NS
L = 16
NCHUNK = 2
BLK = 256


def _logits_body(w_ref, x_ref, out_ref, s_ref):
    lg = lax.dot_general(
        w_ref[...], x_ref[...],
        dimension_numbers=(((1,), (1,)), ((), ())),
        preferred_element_type=jnp.float32,
    )
    out_ref[0] = lg
    m = jnp.max(lg, axis=0)
    s_ref[0, 0] = jnp.sum(jnp.exp(lg - m[None, :]), axis=0)


def _make_router(E, TPW):
    mesh = plsc.VectorSubcoreMesh(
        core_axis_name="c", subcore_axis_name="s", num_cores=NC, num_subcores=NS
    )

    @functools.partial(
        pl.kernel,
        out_type=[
            jax.ShapeDtypeStruct((NW, TPW), jnp.float32),
            jax.ShapeDtypeStruct((NW, TPW), jnp.float32),
            jax.ShapeDtypeStruct((NW, TPW), jnp.int32),
            jax.ShapeDtypeStruct((NW, TPW), jnp.int32),
        ],
        mesh=mesh,
        scratch_types=[
            pltpu.VMEM((E, TPW), jnp.float32),
            pltpu.VMEM((1, TPW), jnp.float32),
            pltpu.VMEM((TPW,), jnp.float32),
            pltpu.VMEM((TPW,), jnp.float32),
            pltpu.VMEM((TPW,), jnp.int32),
            pltpu.VMEM((TPW,), jnp.int32),
        ],
    )
    def router(lg_hbm, s_hbm, w1_hbm, w2_hbm, i1_hbm, i2_hbm,
               lg_v, s_v, w1_v, w2_v, i1_v, i2_v):
        wid = lax.axis_index("s") * NC + lax.axis_index("c")
        pltpu.sync_copy(lg_hbm.at[wid], lg_v)
        pltpu.sync_copy(s_hbm.at[wid], s_v)

        def chunk(c, carry):
            off = c * L
            m1 = lg_v[0, pl.ds(off, L)]
            i1 = jnp.zeros((L,), jnp.int32)
            m2 = jnp.full((L,), -jnp.inf, jnp.float32)
            i2 = jnp.zeros((L,), jnp.int32)
            for e in range(1, E):
                v = lg_v[e, pl.ds(off, L)]
                e_vec = jnp.full((L,), e, jnp.int32)
                gt1 = v > m1
                gt2 = v > m2
                i2 = jnp.where(gt1, i1, jnp.where(gt2, e_vec, i2))
                m2 = jnp.maximum(m2, jnp.minimum(m1, v))
                i1 = jnp.where(gt1, e_vec, i1)
                m1 = jnp.maximum(m1, v)
            r = 1.0 / s_v[0, pl.ds(off, L)]
            w1_v[pl.ds(off, L)] = r
            w2_v[pl.ds(off, L)] = jnp.exp(m2 - m1) * r
            i1_v[pl.ds(off, L)] = i1
            i2_v[pl.ds(off, L)] = i2
            return carry

        lax.fori_loop(0, TPW // L, chunk, 0)
        pltpu.sync_copy(w1_v, w1_hbm.at[wid])
        pltpu.sync_copy(w2_v, w2_hbm.at[wid])
        pltpu.sync_copy(i1_v, i1_hbm.at[wid])
        pltpu.sync_copy(i2_v, i2_hbm.at[wid])

    return router


def kernel(x, W):
    T = x.shape[0] * x.shape[1]
    HS = x.shape[2]
    E = W.shape[0]
    TC = T // NCHUNK
    TPW = TC // NW
    G = TC // BLK
    BPW = TPW // BLK
    x_flat = x.reshape(T, HS)
    router = _make_router(E, TPW)

    parts = []
    for k in range(NCHUNK):
        base = k * G

        logits, denom = pl.pallas_call(
            _logits_body,
            grid=(G,),
            in_specs=[
                pl.BlockSpec((E, HS), lambda i: (0, 0)),
                pl.BlockSpec((BLK, HS), lambda i, b=base: (b + i, 0)),
            ],
            out_specs=[
                pl.BlockSpec((1, E, BLK), lambda i: (i, 0, 0)),
                pl.BlockSpec((1, 1, BLK), lambda i: (i, 0, 0)),
            ],
            out_shape=[
                jax.ShapeDtypeStruct((G, E, BLK), jnp.float32),
                jax.ShapeDtypeStruct((G, 1, BLK), jnp.float32),
            ],
        )(W, x_flat)

        lg_w = logits.reshape(NW, BPW, E, BLK).swapaxes(1, 2).reshape(NW, E, TPW)
        s_w = denom.reshape(NW, 1, TPW)
        parts.append(router(lg_w, s_w))

    w1 = jnp.concatenate([p[0].reshape(TC) for p in parts])
    w2 = jnp.concatenate([p[1].reshape(TC) for p in parts])
    i1 = jnp.concatenate([p[2].reshape(TC) for p in parts])
    i2 = jnp.concatenate([p[3].reshape(TC) for p in parts])
    expert_weights = jnp.stack([w1, w2], axis=-1)
    expert_indices = jnp.stack([i1, i2], axis=-1)
    return expert_weights, expert_indices.astype(jnp.int64)


# speedup vs baseline: 2.6474x; 2.6474x over previous
import functools

import jax
import jax.numpy as jnp
from jax import lax
from jax.experimental import pallas as pl
from jax.experimental.pallas import tpu as pltpu
from jax.experimental.pallas import tpu_sc as plsc

NC = 2
NS = 16
NW = NC * NS
L = 16
SB = 128


def _logits_body(w_ref, x_ref, out_ref, s_ref):
    sb = x_ref.shape[0]
    nt = x_ref.shape[1] // 4
    b = sb * 4
    xb = (
        x_ref[...]
        .reshape(sb, nt, 4, 128)
        .swapaxes(1, 2)
        .reshape(b, nt * 128)
    )
    lg = lax.dot_general(
        w_ref[...], xb,
        dimension_numbers=(((1,), (1,)), ((), ())),
        preferred_element_type=jnp.float32,
    )
    out_ref[0] = lg
    m = jnp.max(lg, axis=0)
    s_ref[0, 0] = jnp.sum(jnp.exp(lg - m[None, :]), axis=0)


def _make_router(E, TPW):
    mesh = plsc.VectorSubcoreMesh(
        core_axis_name="c", subcore_axis_name="s", num_cores=NC, num_subcores=NS
    )

    @functools.partial(
        pl.kernel,
        out_type=[
            jax.ShapeDtypeStruct((NW, TPW), jnp.float32),
            jax.ShapeDtypeStruct((NW, TPW), jnp.float32),
            jax.ShapeDtypeStruct((NW, TPW), jnp.int32),
            jax.ShapeDtypeStruct((NW, TPW), jnp.int32),
        ],
        mesh=mesh,
        scratch_types=[
            pltpu.VMEM((E, TPW), jnp.float32),
            pltpu.VMEM((1, TPW), jnp.float32),
            pltpu.VMEM((TPW,), jnp.float32),
            pltpu.VMEM((TPW,), jnp.float32),
            pltpu.VMEM((TPW,), jnp.int32),
            pltpu.VMEM((TPW,), jnp.int32),
        ],
    )
    def router(lg_hbm, s_hbm, w1_hbm, w2_hbm, i1_hbm, i2_hbm,
               lg_v, s_v, w1_v, w2_v, i1_v, i2_v):
        wid = lax.axis_index("s") * NC + lax.axis_index("c")
        pltpu.sync_copy(lg_hbm.at[wid], lg_v)
        pltpu.sync_copy(s_hbm.at[wid], s_v)

        def chunk(c, carry):
            off = c * L
            m1 = lg_v[0, pl.ds(off, L)]
            i1 = jnp.zeros((L,), jnp.int32)
            m2 = jnp.full((L,), -jnp.inf, jnp.float32)
            i2 = jnp.zeros((L,), jnp.int32)
            for e in range(1, E):
                v = lg_v[e, pl.ds(off, L)]
                e_vec = jnp.full((L,), e, jnp.int32)
                gt1 = v > m1
                gt2 = v > m2
                i2 = jnp.where(gt1, i1, jnp.where(gt2, e_vec, i2))
                m2 = jnp.maximum(m2, jnp.minimum(m1, v))
                i1 = jnp.where(gt1, e_vec, i1)
                m1 = jnp.maximum(m1, v)
            r = 1.0 / s_v[0, pl.ds(off, L)]
            w1_v[pl.ds(off, L)] = r
            w2_v[pl.ds(off, L)] = jnp.exp(m2 - m1) * r
            i1_v[pl.ds(off, L)] = i1
            i2_v[pl.ds(off, L)] = i2
            return carry

        lax.fori_loop(0, TPW // L, chunk, 0)
        pltpu.sync_copy(w1_v, w1_hbm.at[wid])
        pltpu.sync_copy(w2_v, w2_hbm.at[wid])
        pltpu.sync_copy(i1_v, i1_hbm.at[wid])
        pltpu.sync_copy(i2_v, i2_hbm.at[wid])

    return router


def kernel(x, W):
    SL, BS, HS = x.shape
    T = SL * BS
    E = W.shape[0]
    NT = HS // 128
    TPW = T // NW
    BT = SB * BS
    G = SL // SB

    xv = x.reshape(SL, BS, NT, 128).transpose(0, 2, 1, 3).reshape(SL, NT * BS, 128)

    logits, denom = pl.pallas_call(
        _logits_body,
        grid=(G,),
        in_specs=[
            pl.BlockSpec((E, HS), lambda i: (0, 0)),
            pl.BlockSpec((SB, NT * BS, 128), lambda i: (i, 0, 0)),
        ],
        out_specs=[
            pl.BlockSpec((1, E, BT), lambda i: (i, 0, 0)),
            pl.BlockSpec((1, 1, BT), lambda i: (i, 0, 0)),
        ],
        out_shape=[
            jax.ShapeDtypeStruct((G, E, BT), jnp.float32),
            jax.ShapeDtypeStruct((G, 1, BT), jnp.float32),
        ],
    )(W, xv)

    lg_w = logits.reshape(NW, E, TPW)
    s_w = denom.reshape(NW, 1, TPW)
    w1, w2, i1, i2 = _make_router(E, TPW)(lg_w, s_w)
    expert_weights = jnp.stack([w1.reshape(T), w2.reshape(T)], axis=-1)
    expert_indices = jnp.stack([i1.reshape(T), i2.reshape(T)], axis=-1)
    return expert_weights, expert_indices.astype(jnp.int64)


# speedup vs baseline: 2.9241x; 1.1045x over previous
import functools

import jax
import jax.numpy as jnp
from jax import lax
from jax.experimental import pallas as pl
from jax.experimental.pallas import tpu as pltpu
from jax.experimental.pallas import tpu_sc as plsc

NC = 2
NS = 16
NW = NC * NS
L = 16
SB = 256


def _logits_body(w_ref, x_ref, out_ref, s_ref):
    sb = x_ref.shape[0]
    nt = x_ref.shape[1] // 4
    b = sb * 4
    wpb = out_ref.shape[0]
    tpw = b // wpb
    xb = (
        x_ref[...]
        .reshape(sb, nt, 4, 128)
        .swapaxes(1, 2)
        .reshape(b, nt * 128)
    )
    lg = lax.dot_general(
        w_ref[...], xb,
        dimension_numbers=(((1,), (1,)), ((), ())),
        preferred_element_type=jnp.float32,
    )
    m = jnp.max(lg, axis=0)
    s = jnp.sum(jnp.exp(lg - m[None, :]), axis=0)
    for j in range(wpb):
        out_ref[j] = lg[:, j * tpw:(j + 1) * tpw]
        s_ref[j, 0] = s[j * tpw:(j + 1) * tpw]


def _make_router(E, TPW):
    mesh = plsc.VectorSubcoreMesh(
        core_axis_name="c", subcore_axis_name="s", num_cores=NC, num_subcores=NS
    )

    @functools.partial(
        pl.kernel,
        out_type=[
            jax.ShapeDtypeStruct((2, NW, TPW), jnp.float32),
            jax.ShapeDtypeStruct((2, NW, TPW), jnp.int32),
        ],
        mesh=mesh,
        scratch_types=[
            pltpu.VMEM((E, TPW), jnp.float32),
            pltpu.VMEM((1, TPW), jnp.float32),
            pltpu.VMEM((TPW,), jnp.float32),
            pltpu.VMEM((TPW,), jnp.float32),
            pltpu.VMEM((TPW,), jnp.int32),
            pltpu.VMEM((TPW,), jnp.int32),
        ],
    )
    def router(lg_hbm, s_hbm, w_hbm, i_hbm,
               lg_v, s_v, w1_v, w2_v, i1_v, i2_v):
        wid = lax.axis_index("s") * NC + lax.axis_index("c")
        pltpu.sync_copy(lg_hbm.at[wid], lg_v)
        pltpu.sync_copy(s_hbm.at[wid], s_v)

        def chunk(c, carry):
            off = c * L
            m1 = lg_v[0, pl.ds(off, L)]
            i1 = jnp.zeros((L,), jnp.int32)
            m2 = jnp.full((L,), -jnp.inf, jnp.float32)
            i2 = jnp.zeros((L,), jnp.int32)
            for e in range(1, E):
                v = lg_v[e, pl.ds(off, L)]
                e_vec = jnp.full((L,), e, jnp.int32)
                gt1 = v > m1
                gt2 = v > m2
                i2 = jnp.where(gt1, i1, jnp.where(gt2, e_vec, i2))
                m2 = jnp.maximum(m2, jnp.minimum(m1, v))
                i1 = jnp.where(gt1, e_vec, i1)
                m1 = jnp.maximum(m1, v)
            r = 1.0 / s_v[0, pl.ds(off, L)]
            w1_v[pl.ds(off, L)] = r
            w2_v[pl.ds(off, L)] = jnp.exp(m2 - m1) * r
            i1_v[pl.ds(off, L)] = i1
            i2_v[pl.ds(off, L)] = i2
            return carry

        lax.fori_loop(0, TPW // L, chunk, 0)
        pltpu.sync_copy(w1_v, w_hbm.at[0, wid])
        pltpu.sync_copy(w2_v, w_hbm.at[1, wid])
        pltpu.sync_copy(i1_v, i_hbm.at[0, wid])
        pltpu.sync_copy(i2_v, i_hbm.at[1, wid])

    return router


def kernel(x, W):
    SL, BS, HS = x.shape
    T = SL * BS
    E = W.shape[0]
    NT = HS // 128
    TPW = T // NW
    BT = SB * BS
    WPB = BT // TPW
    G = SL // SB

    xv = x.reshape(SL, BS, NT, 128).transpose(0, 2, 1, 3).reshape(SL, NT * BS, 128)

    logits, denom = pl.pallas_call(
        _logits_body,
        grid=(G,),
        in_specs=[
            pl.BlockSpec((E, HS), lambda i: (0, 0)),
            pl.BlockSpec((SB, NT * BS, 128), lambda i: (i, 0, 0)),
        ],
        out_specs=[
            pl.BlockSpec((WPB, E, TPW), lambda i: (i, 0, 0)),
            pl.BlockSpec((WPB, 1, TPW), lambda i: (i, 0, 0)),
        ],
        out_shape=[
            jax.ShapeDtypeStruct((NW, E, TPW), jnp.float32),
            jax.ShapeDtypeStruct((NW, 1, TPW), jnp.float32),
        ],
    )(W, xv)

    w, idx = _make_router(E, TPW)(logits, denom)
    expert_weights = w.reshape(2, T).T
    expert_indices = idx.reshape(2, T).T
    return expert_weights, expert_indices.astype(jnp.int64)


# speedup vs baseline: 3.0153x; 1.0312x over previous
import functools

import jax
import jax.numpy as jnp
from jax import lax
from jax.experimental import pallas as pl
from jax.experimental.pallas import tpu as pltpu
from jax.experimental.pallas import tpu_sc as plsc

NC = 2
NS = 16
NW = NC * NS
L = 16
SB = 512


def _logits_body(w_ref, x_ref, out_ref, s_ref):
    sb = x_ref.shape[0]
    nt = x_ref.shape[1] // 4
    b = sb * 4
    wpb = out_ref.shape[0]
    tpw = b // wpb
    xb = (
        x_ref[...]
        .reshape(sb, nt, 4, 128)
        .swapaxes(1, 2)
        .reshape(b, nt * 128)
    )
    lg = lax.dot_general(
        w_ref[...], xb,
        dimension_numbers=(((1,), (1,)), ((), ())),
        preferred_element_type=jnp.float32,
    )
    m = jnp.max(lg, axis=0)
    s = jnp.sum(jnp.exp(lg - m[None, :]), axis=0)
    for j in range(wpb):
        out_ref[j] = lg[:, j * tpw:(j + 1) * tpw]
        s_ref[j, 0] = s[j * tpw:(j + 1) * tpw]


def _make_router(E, TPW):
    mesh = plsc.VectorSubcoreMesh(
        core_axis_name="c", subcore_axis_name="s", num_cores=NC, num_subcores=NS
    )

    @functools.partial(
        pl.kernel,
        out_type=[
            jax.ShapeDtypeStruct((2, NW, TPW), jnp.float32),
            jax.ShapeDtypeStruct((2, NW, TPW), jnp.int32),
        ],
        mesh=mesh,
        scratch_types=[
            pltpu.VMEM((E, TPW), jnp.float32),
            pltpu.VMEM((1, TPW), jnp.float32),
            pltpu.VMEM((TPW,), jnp.float32),
            pltpu.VMEM((TPW,), jnp.float32),
            pltpu.VMEM((TPW,), jnp.int32),
            pltpu.VMEM((TPW,), jnp.int32),
        ],
    )
    def router(lg_hbm, s_hbm, w_hbm, i_hbm,
               lg_v, s_v, w1_v, w2_v, i1_v, i2_v):
        wid = lax.axis_index("s") * NC + lax.axis_index("c")
        pltpu.sync_copy(lg_hbm.at[wid], lg_v)
        pltpu.sync_copy(s_hbm.at[wid], s_v)

        def chunk(c, carry):
            offs = (c * (2 * L), c * (2 * L) + L)
            m1 = [lg_v[0, pl.ds(o, L)] for o in offs]
            i1 = [jnp.zeros((L,), jnp.int32) for _ in offs]
            m2 = [jnp.full((L,), -jnp.inf, jnp.float32) for _ in offs]
            i2 = [jnp.zeros((L,), jnp.int32) for _ in offs]
            for e in range(1, E):
                e_vec = jnp.full((L,), e, jnp.int32)
                for k, o in enumerate(offs):
                    v = lg_v[e, pl.ds(o, L)]
                    gt1 = v > m1[k]
                    gt2 = v > m2[k]
                    i2[k] = jnp.where(gt1, i1[k], jnp.where(gt2, e_vec, i2[k]))
                    m2[k] = jnp.maximum(m2[k], jnp.minimum(m1[k], v))
                    i1[k] = jnp.where(gt1, e_vec, i1[k])
                    m1[k] = jnp.maximum(m1[k], v)
            for k, o in enumerate(offs):
                r = 1.0 / s_v[0, pl.ds(o, L)]
                w1_v[pl.ds(o, L)] = r
                w2_v[pl.ds(o, L)] = jnp.exp(m2[k] - m1[k]) * r
                i1_v[pl.ds(o, L)] = i1[k]
                i2_v[pl.ds(o, L)] = i2[k]
            return carry

        lax.fori_loop(0, TPW // (2 * L), chunk, 0)
        pltpu.sync_copy(w1_v, w_hbm.at[0, wid])
        pltpu.sync_copy(w2_v, w_hbm.at[1, wid])
        pltpu.sync_copy(i1_v, i_hbm.at[0, wid])
        pltpu.sync_copy(i2_v, i_hbm.at[1, wid])

    return router


def kernel(x, W):
    SL, BS, HS = x.shape
    T = SL * BS
    E = W.shape[0]
    NT = HS // 128
    TPW = T // NW
    BT = SB * BS
    WPB = BT // TPW
    G = SL // SB

    xv = x.reshape(SL, BS, NT, 128).transpose(0, 2, 1, 3).reshape(SL, NT * BS, 128)

    logits, denom = pl.pallas_call(
        _logits_body,
        grid=(G,),
        in_specs=[
            pl.BlockSpec((E, HS), lambda i: (0, 0)),
            pl.BlockSpec((SB, NT * BS, 128), lambda i: (i, 0, 0)),
        ],
        out_specs=[
            pl.BlockSpec((WPB, E, TPW), lambda i: (i, 0, 0)),
            pl.BlockSpec((WPB, 1, TPW), lambda i: (i, 0, 0)),
        ],
        out_shape=[
            jax.ShapeDtypeStruct((NW, E, TPW), jnp.float32),
            jax.ShapeDtypeStruct((NW, 1, TPW), jnp.float32),
        ],
    )(W, xv)

    w, idx = _make_router(E, TPW)(logits, denom)
    expert_weights = w.reshape(2, T).T
    expert_indices = idx.reshape(2, T).T
    return expert_weights, expert_indices.astype(jnp.int64)
